# baseline (speedup 1.0000x reference)
import jax
import jax.numpy as jnp
from jax import lax
from jax.experimental import pallas as pl
from jax.experimental.pallas import tpu as pltpu
from jax.experimental.pallas import tpu_sc as plsc

DM = 16
B = 4
T = 2048
D = 1024
L = T + DM
CH = 16
RING = 64
W8ROWS = 256
NSUB = 16
GROUPS = D // 16


def _delay_kernel(x_hbm, d_hbm, out_hbm, ring, ob0, ob1, dly,
                  isem0, isem1, osem0, osem1):
    cid = lax.axis_index("c")
    sid = lax.axis_index("s")
    wid = cid * 16 + sid
    b = wid // 8
    w8 = wid % 8
    base = w8 * W8ROWS
    xrow = b * T
    orow = b * L
    last = w8 == 7

    pltpu.sync_copy(d_hbm, dly)
    lanes = lax.iota(jnp.int32, 16)
    zeros16 = jnp.zeros((16,), jnp.float32)

    def zero_slots(slot0):
        def zbody(i, _):
            for g in range(GROUPS):
                ring[slot0 + i, pl.ds(g * 16, 16)] = zeros16
            return 0
        lax.fori_loop(0, CH, zbody, 0)

    def stage(row, sem):
        slot = lax.rem(row, RING)
        return pltpu.make_async_copy(
            x_hbm.at[pl.ds(xrow + row, CH), :],
            ring.at[pl.ds(slot, CH), :],
            sem,
        )

    def out_dma(ob, t0, sem):
        return pltpu.make_async_copy(
            ob, out_hbm.at[pl.ds(orow + t0, CH), :], sem)

    isems = [isem0, isem1]
    osems = [osem0, osem1]
    obufs = [ob0, ob1]

    @pl.when(w8 == 0)
    def _():
        zero_slots(RING - CH)

    @pl.when(w8 != 0)
    def _():
        hslot = lax.rem(base - CH, RING)
        pltpu.sync_copy(
            x_hbm.at[pl.ds(xrow + base - CH, CH), :],
            ring.at[pl.ds(hslot, CH), :],
        )

    stage(base, isems[0]).start()
    stage(base + CH, isems[1]).start()

    def run_sub(k):
        t0 = base + k * CH
        sel = k % 2
        ob = obufs[sel]

        if k < NSUB:
            stage(t0, isems[sel]).wait()
        else:
            zero_slots(0)

        if k + 2 < NSUB:
            stage(t0 + 2 * CH, isems[sel]).start()

        if k >= 2:
            out_dma(ob, t0 - 2 * CH, osems[sel]).wait()

        @plsc.parallel_loop(0, GROUPS, unroll=8)
        def grp_body(g):
            goff = g * 16
            dv = dly[pl.ds(goff, 16)]
            cv = goff + lanes
            for r in range(CH):
                rv = ((t0 + r) - dv) & (RING - 1)
                ob[r, pl.ds(goff, 16)] = plsc.load_gather(ring, [rv, cv])

        out_dma(ob, t0, osems[sel]).start()

    for k in range(NSUB):
        run_sub(k)

    @pl.when(last)
    def _():
        run_sub(NSUB)

    out_dma(obufs[0], base + (NSUB - 2) * CH, osems[0]).wait()
    out_dma(obufs[1], base + (NSUB - 1) * CH, osems[1]).wait()


def kernel(x, delays):
    x2 = x.reshape(B * T, D)
    mesh = plsc.VectorSubcoreMesh(core_axis_name="c", subcore_axis_name="s")
    out = pl.kernel(
        _delay_kernel,
        mesh=mesh,
        out_type=jax.ShapeDtypeStruct((B * L, D), jnp.float32),
        scratch_types=[
            pltpu.VMEM((RING, D), jnp.float32),
            pltpu.VMEM((CH, D), jnp.float32),
            pltpu.VMEM((CH, D), jnp.float32),
            pltpu.VMEM((D,), jnp.int32),
            pltpu.SemaphoreType.DMA,
            pltpu.SemaphoreType.DMA,
            pltpu.SemaphoreType.DMA,
            pltpu.SemaphoreType.DMA,
        ],
        compiler_params=pltpu.CompilerParams(needs_layout_passes=False),
    )(x2, delays)
    return out.reshape(B, L, D)


# speedup vs baseline: 1.1576x; 1.1576x over previous
import jax
import jax.numpy as jnp
from jax import lax
from jax.experimental import pallas as pl
from jax.experimental.pallas import tpu as pltpu
from jax.experimental.pallas import tpu_sc as plsc

DM = 16
B = 4
T = 2048
D = 1024
L = T + DM
CH = 16
RING = 64
W8ROWS = 256
NSUB = 16
GROUPS = D // 16


def _delay_kernel(x_hbm, d_hbm, out_hbm, ring, ob0, ob1, dly,
                  isem0, isem1, osem0, osem1):
    cid = lax.axis_index("c")
    sid = lax.axis_index("s")
    wid = cid * 16 + sid
    b = wid // 8
    w8 = wid % 8
    base = w8 * W8ROWS
    xrow = b * T
    orow = b * L
    last = w8 == 7

    pltpu.sync_copy(d_hbm, dly)
    lanes = lax.iota(jnp.int32, 16)
    zeros16 = jnp.zeros((16,), jnp.float32)

    def zero_slots(slot0):
        def zbody(i, _):
            for g in range(GROUPS):
                ring[slot0 + i, pl.ds(g * 16, 16)] = zeros16
            return 0
        lax.fori_loop(0, CH, zbody, 0)

    def stage(row, sem):
        slot = lax.rem(row, RING)
        return pltpu.make_async_copy(
            x_hbm.at[pl.ds(xrow + row, CH), :],
            ring.at[pl.ds(slot, CH), :],
            sem,
        )

    def out_dma(ob, t0, sem):
        return pltpu.make_async_copy(
            ob, out_hbm.at[pl.ds(orow + t0, CH), :], sem)

    isems = [isem0, isem1]
    osems = [osem0, osem1]
    obufs = [ob0, ob1]

    @pl.when(w8 == 0)
    def _():
        zero_slots(RING - CH)

    @pl.when(w8 != 0)
    def _():
        hslot = lax.rem(base - CH, RING)
        pltpu.sync_copy(
            x_hbm.at[pl.ds(xrow + base - CH, CH), :],
            ring.at[pl.ds(hslot, CH), :],
        )

    stage(base, isems[0]).start()
    stage(base + CH, isems[1]).start()

    def run_sub(k):
        t0 = base + k * CH
        sel = k % 2
        ob = obufs[sel]

        if k < NSUB:
            stage(t0, isems[sel]).wait()
        else:
            zero_slots(0)

        if k + 2 < NSUB:
            stage(t0 + 2 * CH, isems[sel]).start()

        if k >= 2:
            out_dma(ob, t0 - 2 * CH, osems[sel]).wait()

        @plsc.parallel_loop(0, GROUPS, unroll=2)
        def grp_body(g):
            goff = g * 16
            dv = dly[pl.ds(goff, 16)]
            cv = goff + lanes
            for r in range(CH):
                rv = ((t0 + r) - dv) & (RING - 1)
                ob[r, pl.ds(goff, 16)] = plsc.load_gather(ring, [rv, cv])

        out_dma(ob, t0, osems[sel]).start()

    for k in range(NSUB):
        run_sub(k)

    @pl.when(last)
    def _():
        run_sub(NSUB)

    out_dma(obufs[0], base + (NSUB - 2) * CH, osems[0]).wait()
    out_dma(obufs[1], base + (NSUB - 1) * CH, osems[1]).wait()


def kernel(x, delays):
    x2 = x.reshape(B * T, D)
    mesh = plsc.VectorSubcoreMesh(core_axis_name="c", subcore_axis_name="s")
    out = pl.kernel(
        _delay_kernel,
        mesh=mesh,
        out_type=jax.ShapeDtypeStruct((B * L, D), jnp.float32),
        scratch_types=[
            pltpu.VMEM((RING, D), jnp.float32),
            pltpu.VMEM((CH, D), jnp.float32),
            pltpu.VMEM((CH, D), jnp.float32),
            pltpu.VMEM((D,), jnp.int32),
            pltpu.SemaphoreType.DMA,
            pltpu.SemaphoreType.DMA,
            pltpu.SemaphoreType.DMA,
            pltpu.SemaphoreType.DMA,
        ],
        compiler_params=pltpu.CompilerParams(needs_layout_passes=False),
    )(x2, delays)
    return out.reshape(B, L, D)


# speedup vs baseline: 1.1861x; 1.0246x over previous
import jax
import jax.numpy as jnp
from jax import lax
from jax.experimental import pallas as pl
from jax.experimental.pallas import tpu as pltpu
from jax.experimental.pallas import tpu_sc as plsc

DM = 16
B = 4
T = 2048
D = 1024
L = T + DM
CH = 16
RING = 64
W8ROWS = 256
NSUB = 16
GROUPS = D // 16


def _delay_kernel(x_hbm, d_hbm, out_hbm, ring, ob0, ob1, dly,
                  isem0, isem1, osem0, osem1):
    cid = lax.axis_index("c")
    sid = lax.axis_index("s")
    wid = cid * 16 + sid
    b = wid // 8
    w8 = wid % 8
    base = w8 * W8ROWS
    xrow = b * T
    orow = b * L
    last = w8 == 7

    pltpu.sync_copy(d_hbm, dly)
    lanes = lax.iota(jnp.int32, 16)
    zeros16 = jnp.zeros((16,), jnp.float32)

    def zero_slots(slot0):
        def zbody(i, _):
            for g in range(GROUPS):
                ring[slot0 + i, pl.ds(g * 16, 16)] = zeros16
            return 0
        lax.fori_loop(0, CH, zbody, 0)

    def stage(row, sem):
        slot = lax.rem(row, RING)
        return pltpu.make_async_copy(
            x_hbm.at[pl.ds(xrow + row, CH), :],
            ring.at[pl.ds(slot, CH), :],
            sem,
        )

    def out_dma(ob, t0, sem):
        return pltpu.make_async_copy(
            ob, out_hbm.at[pl.ds(orow + t0, CH), :], sem)

    isems = [isem0, isem1]
    osems = [osem0, osem1]
    obufs = [ob0, ob1]

    @pl.when(w8 == 0)
    def _():
        zero_slots(RING - CH)

    @pl.when(w8 != 0)
    def _():
        hslot = lax.rem(base - CH, RING)
        pltpu.sync_copy(
            x_hbm.at[pl.ds(xrow + base - CH, CH), :],
            ring.at[pl.ds(hslot, CH), :],
        )

    stage(base, isems[0]).start()
    stage(base + CH, isems[1]).start()

    def run_sub(k):
        t0 = base + k * CH
        sel = k % 2
        ob = obufs[sel]

        if k < NSUB:
            stage(t0, isems[sel]).wait()
        else:
            zero_slots(0)

        if k + 2 < NSUB:
            stage(t0 + 2 * CH, isems[sel]).start()

        if k >= 2:
            out_dma(ob, t0 - 2 * CH, osems[sel]).wait()

        @plsc.parallel_loop(0, GROUPS, unroll=1)
        def grp_body(g):
            goff = g * 16
            dv = dly[pl.ds(goff, 16)]
            cv = goff + lanes
            for r in range(CH):
                rv = ((t0 + r) - dv) & (RING - 1)
                ob[r, pl.ds(goff, 16)] = plsc.load_gather(ring, [rv, cv])

        out_dma(ob, t0, osems[sel]).start()

    for k in range(NSUB):
        run_sub(k)

    @pl.when(last)
    def _():
        run_sub(NSUB)

    out_dma(obufs[0], base + (NSUB - 2) * CH, osems[0]).wait()
    out_dma(obufs[1], base + (NSUB - 1) * CH, osems[1]).wait()


def kernel(x, delays):
    x2 = x.reshape(B * T, D)
    mesh = plsc.VectorSubcoreMesh(core_axis_name="c", subcore_axis_name="s")
    out = pl.kernel(
        _delay_kernel,
        mesh=mesh,
        out_type=jax.ShapeDtypeStruct((B * L, D), jnp.float32),
        scratch_types=[
            pltpu.VMEM((RING, D), jnp.float32),
            pltpu.VMEM((CH, D), jnp.float32),
            pltpu.VMEM((CH, D), jnp.float32),
            pltpu.VMEM((D,), jnp.int32),
            pltpu.SemaphoreType.DMA,
            pltpu.SemaphoreType.DMA,
            pltpu.SemaphoreType.DMA,
            pltpu.SemaphoreType.DMA,
        ],
        compiler_params=pltpu.CompilerParams(needs_layout_passes=False),
    )(x2, delays)
    return out.reshape(B, L, D)


# speedup vs baseline: 1.2026x; 1.0139x over previous
import jax
import jax.numpy as jnp
from jax import lax
from jax.experimental import pallas as pl
from jax.experimental.pallas import tpu as pltpu
from jax.experimental.pallas import tpu_sc as plsc

DM = 16
B = 4
T = 2048
D = 1024
L = T + DM
CH = 16
RING = 64
W8ROWS = 256
NSUB = 16
GROUPS = D // 16


def _delay_kernel(x_hbm, d_hbm, out_hbm, ring, ob0, ob1, dly,
                  isem0, isem1, osem0, osem1):
    cid = lax.axis_index("c")
    sid = lax.axis_index("s")
    wid = cid * 16 + sid
    b = wid // 8
    w8 = wid % 8
    base = w8 * W8ROWS
    xrow = b * T
    orow = b * L
    last = w8 == 7

    lanes = lax.iota(jnp.int32, 16)
    zeros16 = jnp.zeros((16,), jnp.float32)

    def zero_slots(slot0):
        def zbody(i, _):
            for g in range(GROUPS):
                ring[slot0 + i, pl.ds(g * 16, 16)] = zeros16
            return 0
        lax.fori_loop(0, CH, zbody, 0)

    def stage(row, sem):
        slot = lax.rem(row, RING)
        return pltpu.make_async_copy(
            x_hbm.at[pl.ds(xrow + row, CH), :],
            ring.at[pl.ds(slot, CH), :],
            sem,
        )

    def out_dma(ob, t0, sem):
        return pltpu.make_async_copy(
            ob, out_hbm.at[pl.ds(orow + t0, CH), :], sem)

    isems = [isem0, isem1]
    osems = [osem0, osem1]
    obufs = [ob0, ob1]

    stage(base, isems[0]).start()
    stage(base + CH, isems[1]).start()

    def halo_dma():
        hslot = lax.rem(base - CH, RING)
        return pltpu.make_async_copy(
            x_hbm.at[pl.ds(xrow + base - CH, CH), :],
            ring.at[pl.ds(hslot, CH), :],
            osem0,
        )

    @pl.when(w8 == 0)
    def _():
        zero_slots(RING - CH)

    @pl.when(w8 != 0)
    def _():
        halo_dma().start()

    pltpu.sync_copy(d_hbm, dly)

    @pl.when(w8 != 0)
    def _():
        halo_dma().wait()

    def run_sub(k):
        t0 = base + k * CH
        sel = k % 2
        ob = obufs[sel]

        if k < NSUB:
            stage(t0, isems[sel]).wait()
        else:
            zero_slots(0)

        if k + 2 < NSUB:
            stage(t0 + 2 * CH, isems[sel]).start()

        if k >= 2:
            out_dma(ob, t0 - 2 * CH, osems[sel]).wait()

        @plsc.parallel_loop(0, GROUPS, unroll=1)
        def grp_body(g):
            goff = g * 16
            dv = dly[pl.ds(goff, 16)]
            cv = goff + lanes
            for r in range(CH):
                rv = ((t0 + r) - dv) & (RING - 1)
                ob[r, pl.ds(goff, 16)] = plsc.load_gather(ring, [rv, cv])

        out_dma(ob, t0, osems[sel]).start()

    for k in range(NSUB):
        run_sub(k)

    @pl.when(last)
    def _():
        run_sub(NSUB)

    out_dma(obufs[0], base + (NSUB - 2) * CH, osems[0]).wait()
    out_dma(obufs[1], base + (NSUB - 1) * CH, osems[1]).wait()


def kernel(x, delays):
    x2 = x.reshape(B * T, D)
    mesh = plsc.VectorSubcoreMesh(core_axis_name="c", subcore_axis_name="s")
    out = pl.kernel(
        _delay_kernel,
        mesh=mesh,
        out_type=jax.ShapeDtypeStruct((B * L, D), jnp.float32),
        scratch_types=[
            pltpu.VMEM((RING, D), jnp.float32),
            pltpu.VMEM((CH, D), jnp.float32),
            pltpu.VMEM((CH, D), jnp.float32),
            pltpu.VMEM((D,), jnp.int32),
            pltpu.SemaphoreType.DMA,
            pltpu.SemaphoreType.DMA,
            pltpu.SemaphoreType.DMA,
            pltpu.SemaphoreType.DMA,
        ],
        compiler_params=pltpu.CompilerParams(needs_layout_passes=False),
    )(x2, delays)
    return out.reshape(B, L, D)


# speedup vs baseline: 1.2063x; 1.0031x over previous
import jax
import jax.numpy as jnp
from jax import lax
from jax.experimental import pallas as pl
from jax.experimental.pallas import tpu as pltpu
from jax.experimental.pallas import tpu_sc as plsc

DM = 16
B = 4
T = 2048
D = 1024
L = T + DM
CH = 16
RING = 64
W8ROWS = 256
NSUB = 16
GROUPS = D // 16


def _delay_kernel(x_hbm, d_hbm, out_hbm, ring, ob0, ob1, ob2, dly,
                  isem0, isem1, osem0, osem1, osem2):
    cid = lax.axis_index("c")
    sid = lax.axis_index("s")
    wid = cid * 16 + sid
    b = wid // 8
    w8 = wid % 8
    base = w8 * W8ROWS
    xrow = b * T
    orow = b * L
    last = w8 == 7

    lanes = lax.iota(jnp.int32, 16)
    zeros16 = jnp.zeros((16,), jnp.float32)

    def zero_slots(slot0):
        def zbody(i, _):
            for g in range(GROUPS):
                ring[slot0 + i, pl.ds(g * 16, 16)] = zeros16
            return 0
        lax.fori_loop(0, CH, zbody, 0)

    def stage(row, sem):
        slot = lax.rem(row, RING)
        return pltpu.make_async_copy(
            x_hbm.at[pl.ds(xrow + row, CH), :],
            ring.at[pl.ds(slot, CH), :],
            sem,
        )

    def out_dma(ob, t0, sem):
        return pltpu.make_async_copy(
            ob, out_hbm.at[pl.ds(orow + t0, CH), :], sem)

    isems = [isem0, isem1]
    osems = [osem0, osem1, osem2]
    obufs = [ob0, ob1, ob2]

    stage(base, isems[0]).start()
    stage(base + CH, isems[1]).start()

    def halo_dma():
        hslot = lax.rem(base - CH, RING)
        return pltpu.make_async_copy(
            x_hbm.at[pl.ds(xrow + base - CH, CH), :],
            ring.at[pl.ds(hslot, CH), :],
            osem0,
        )

    @pl.when(w8 == 0)
    def _():
        zero_slots(RING - CH)

    @pl.when(w8 != 0)
    def _():
        halo_dma().start()

    pltpu.sync_copy(d_hbm, dly)

    @pl.when(w8 != 0)
    def _():
        halo_dma().wait()

    def run_sub(k):
        t0 = base + k * CH
        sel = k % 2
        osel = k % 3
        ob = obufs[osel]

        if k < NSUB:
            stage(t0, isems[sel]).wait()
        else:
            zero_slots(0)

        if k + 2 < NSUB:
            stage(t0 + 2 * CH, isems[sel]).start()

        if k >= 3:
            out_dma(ob, t0 - 3 * CH, osems[osel]).wait()

        @plsc.parallel_loop(0, GROUPS, unroll=1)
        def grp_body(g):
            goff = g * 16
            dv = dly[pl.ds(goff, 16)]
            cv = goff + lanes
            for r in range(CH):
                rv = ((t0 + r) - dv) & (RING - 1)
                ob[r, pl.ds(goff, 16)] = plsc.load_gather(ring, [rv, cv])

        out_dma(ob, t0, osems[osel]).start()

    for k in range(NSUB):
        run_sub(k)

    @pl.when(last)
    def _():
        run_sub(NSUB)

    out_dma(obufs[0], base + (NSUB - 2) * CH, osems[0]).wait()
    out_dma(obufs[1], base + (NSUB - 1) * CH, osems[1]).wait()
    out_dma(obufs[2], base + (NSUB - 3) * CH, osems[2]).wait()


def kernel(x, delays):
    x2 = x.reshape(B * T, D)
    mesh = plsc.VectorSubcoreMesh(core_axis_name="c", subcore_axis_name="s")
    out = pl.kernel(
        _delay_kernel,
        mesh=mesh,
        out_type=jax.ShapeDtypeStruct((B * L, D), jnp.float32),
        scratch_types=[
            pltpu.VMEM((RING, D), jnp.float32),
            pltpu.VMEM((CH, D), jnp.float32),
            pltpu.VMEM((CH, D), jnp.float32),
            pltpu.VMEM((CH, D), jnp.float32),
            pltpu.VMEM((D,), jnp.int32),
            pltpu.SemaphoreType.DMA,
            pltpu.SemaphoreType.DMA,
            pltpu.SemaphoreType.DMA,
            pltpu.SemaphoreType.DMA,
            pltpu.SemaphoreType.DMA,
        ],
        compiler_params=pltpu.CompilerParams(needs_layout_passes=False),
    )(x2, delays)
    return out.reshape(B, L, D)
